# Initial kernel scaffold; baseline (speedup 1.0000x reference)
import functools
import math

import jax
import jax.numpy as jnp
from jax import lax
from jax.experimental import pallas as pl
from jax.experimental.pallas import tpu as pltpu
from jax.experimental.pallas import tpu_sc as plsc


def _router_body(C, tok_ref, wg_ref, dest1_ref, dest2_ref, w1_ref, w2_ref):
    T, H = tok_ref.shape
    E = wg_ref.shape[0]
    logits = lax.dot_general(
        tok_ref[...].astype(jnp.bfloat16),
        wg_ref[...].astype(jnp.bfloat16),
        (((1,), (1,)), ((), ())),
        preferred_element_type=jnp.float32,
    )
    m = jnp.max(logits, axis=1, keepdims=True)
    ex = jnp.exp(logits - m)
    probs = ex / jnp.sum(ex, axis=1, keepdims=True)

    eio = lax.broadcasted_iota(jnp.int32, (T, E), 1)
    p1 = jnp.max(probs, axis=1, keepdims=True)
    top1 = jnp.min(jnp.where(probs == p1, eio, E), axis=1, keepdims=True)
    mask1 = eio == top1
    pm = jnp.where(mask1, -jnp.inf, probs)
    p2 = jnp.max(pm, axis=1, keepdims=True)
    top2 = jnp.min(jnp.where(pm == p2, eio, E), axis=1, keepdims=True)
    mask2 = eio == top2

    c = jnp.concatenate(
        [mask1.astype(jnp.int32), mask2.astype(jnp.int32)], axis=1
    )
    k = 1
    while k < T:
        shifted = jnp.concatenate(
            [jnp.zeros((k, 2 * E), jnp.int32), c[: T - k, :]], axis=0
        )
        c = c + shifted
        k *= 2
    csum1, csum2 = c[:, :E], c[:, E:]
    cnt1 = c[T - 1 : T, :E]
    cnt2 = c[T - 1 : T, E:]
    rank1 = csum1 - 1
    rank2 = csum2 - 1 + cnt1
    keep1 = mask1 & (rank1 < C)
    keep2 = mask2 & (rank2 < C)

    r1 = jnp.sum(jnp.where(keep1, rank1, 0), axis=1)
    r2 = jnp.sum(jnp.where(keep2, rank2, 0), axis=1)
    k1 = jnp.sum(keep1.astype(jnp.int32), axis=1)
    k2 = jnp.sum(keep2.astype(jnp.int32), axis=1)
    w1_ref[...] = jnp.sum(jnp.where(keep1, probs, 0.0), axis=1)
    w2_ref[...] = jnp.sum(jnp.where(keep2, probs, 0.0), axis=1)

    tot = cnt1 + cnt2
    eio_r = lax.broadcasted_iota(jnp.int32, (1, E), 1)
    ze = jnp.min(jnp.where(tot < C, eio_r, E), axis=1)
    z = jnp.where(ze < E, ze * C + (C - 1), 0)

    t1 = jnp.sum(jnp.where(mask1, eio, 0), axis=1)
    t2 = jnp.sum(jnp.where(mask2, eio, 0), axis=1)
    dest1_ref[...] = jnp.where(k1 > 0, t1 * C + r1, z)
    dest2_ref[...] = jnp.where(k2 > 0, t2 * C + r2, z)


def _router(tokens, W_gate, C):
    T = tokens.shape[0]
    return pl.pallas_call(
        functools.partial(_router_body, C),
        out_shape=[
            jax.ShapeDtypeStruct((T,), jnp.int32),
            jax.ShapeDtypeStruct((T,), jnp.int32),
            jax.ShapeDtypeStruct((T,), jnp.float32),
            jax.ShapeDtypeStruct((T,), jnp.float32),
        ],
    )(tokens, W_gate)


def _make_invert(T, S):
    mesh = plsc.VectorSubcoreMesh(core_axis_name="c", subcore_axis_name="s")

    @functools.partial(
        pl.kernel,
        mesh=mesh,
        out_type=[
            jax.ShapeDtypeStruct((S,), jnp.int32),
            jax.ShapeDtypeStruct((S,), jnp.float32),
        ],
        scratch_types=[
            pltpu.VMEM((T,), jnp.int32),
            pltpu.VMEM((T,), jnp.int32),
            pltpu.VMEM((T,), jnp.float32),
            pltpu.VMEM((T,), jnp.float32),
            pltpu.VMEM((S,), jnp.int32),
            pltpu.VMEM((S,), jnp.float32),
        ],
    )
    def invert(d1_h, d2_h, w1_h, w2_h, src_h, slotw_h,
               d1_v, d2_v, w1_v, w2_v, src_v, slotw_v):
        cid = lax.axis_index("c")
        sid = lax.axis_index("s")

        @pl.when(jnp.logical_and(cid == 0, sid == 0))
        def _():
            pltpu.sync_copy(d1_h, d1_v)
            pltpu.sync_copy(d2_h, d2_v)
            pltpu.sync_copy(w1_h, w1_v)
            pltpu.sync_copy(w2_h, w2_v)

            def init(i, _):
                src_v[pl.ds(i * 16, 16)] = jnp.zeros((16,), jnp.int32)
                slotw_v[pl.ds(i * 16, 16)] = jnp.zeros((16,), jnp.float32)
                return 0

            lax.fori_loop(0, S // 16, init, 0)

            def scat(i, _):
                ids = jnp.full((16,), i * 16, jnp.int32) + lax.iota(jnp.int32, 16)
                d1 = d1_v[pl.ds(i * 16, 16)]
                plsc.store_scatter(src_v, [d1], ids)
                plsc.store_scatter(slotw_v, [d1], w1_v[pl.ds(i * 16, 16)])
                d2 = d2_v[pl.ds(i * 16, 16)]
                plsc.store_scatter(src_v, [d2], ids)
                plsc.store_scatter(slotw_v, [d2], w2_v[pl.ds(i * 16, 16)])
                return 0

            lax.fori_loop(0, T // 16, scat, 0)
            pltpu.sync_copy(src_v, src_h)
            pltpu.sync_copy(slotw_v, slotw_h)

    return invert


def _make_dispatch(T, H, S, nw):
    mesh = plsc.VectorSubcoreMesh(core_axis_name="c", subcore_axis_name="s")
    per_w = S // nw
    chunk = 64

    @functools.partial(
        pl.kernel,
        mesh=mesh,
        out_type=jax.ShapeDtypeStruct((S, H), jnp.float32),
        scratch_types=[
            pltpu.VMEM((chunk,), jnp.int32),
            pltpu.VMEM((chunk, H), jnp.float32),
            pltpu.SemaphoreType.DMA,
        ],
    )
    def dispatch(tok_h, src_h, disp_h, idx_v, rows_v, sem):
        wid = lax.axis_index("s") * 2 + lax.axis_index("c")
        base = wid * per_w
        for ch in range(per_w // chunk):
            off = base + ch * chunk
            pltpu.sync_copy(src_h.at[pl.ds(off, chunk)], idx_v)
            pltpu.async_copy(tok_h.at[idx_v], rows_v, sem).wait()
            pltpu.sync_copy(rows_v, disp_h.at[pl.ds(off, chunk)])

    return dispatch


def _ffn_body(NF, x_ref, wg1_ref, wg2_ref, wu_ref, wd_ref, sw_ref, out_ref,
              acc_ref):
    f = pl.program_id(1)

    @pl.when(f == 0)
    def _():
        acc_ref[...] = jnp.zeros_like(acc_ref)

    x = x_ref[0].astype(jnp.bfloat16)
    cd = (((1,), (1,)), ((), ()))
    g1 = lax.dot_general(x, wg1_ref[0].astype(jnp.bfloat16), cd,
                         preferred_element_type=jnp.float32)
    g2 = lax.dot_general(x, wg2_ref[0].astype(jnp.bfloat16), cd,
                         preferred_element_type=jnp.float32)
    u = lax.dot_general(x, wu_ref[0].astype(jnp.bfloat16), cd,
                        preferred_element_type=jnp.float32)
    a = (g1 * (g2 * jax.nn.sigmoid(g2)) * u).astype(jnp.bfloat16)
    acc_ref[...] += lax.dot_general(a, wd_ref[0].astype(jnp.bfloat16), cd,
                                    preferred_element_type=jnp.float32)

    @pl.when(f == NF - 1)
    def _():
        out_ref[0] = acc_ref[...] * sw_ref[0]


def _ffn(disp, W_gateproj, W_up, W_down, slotw, E, C, H, F):
    Fb = 512
    NF = F // Fb
    x3 = disp.reshape(E, C, H)
    sw3 = slotw.reshape(E, C, 1)
    out = pl.pallas_call(
        functools.partial(_ffn_body, NF),
        grid=(E, NF),
        in_specs=[
            pl.BlockSpec((1, C, H), lambda e, f: (e, 0, 0)),
            pl.BlockSpec((1, Fb, H), lambda e, f: (e, f, 0)),
            pl.BlockSpec((1, Fb, H), lambda e, f, NF=NF: (e, f + NF, 0)),
            pl.BlockSpec((1, Fb, H), lambda e, f: (e, f, 0)),
            pl.BlockSpec((1, H, Fb), lambda e, f: (e, 0, f)),
            pl.BlockSpec((1, C, 1), lambda e, f: (e, 0, 0)),
        ],
        out_specs=pl.BlockSpec((1, C, H), lambda e, f: (e, 0, 0)),
        out_shape=jax.ShapeDtypeStruct((E, C, H), jnp.float32),
        scratch_shapes=[pltpu.VMEM((C, H), jnp.float32)],
    )(x3, W_gateproj, W_gateproj, W_up, W_down, sw3)
    return out.reshape(E * C, H)


def _make_combine(T, H, S, nw):
    mesh = plsc.VectorSubcoreMesh(core_axis_name="c", subcore_axis_name="s")
    per_w = T // nw
    chunk = 32

    @functools.partial(
        pl.kernel,
        mesh=mesh,
        out_type=jax.ShapeDtypeStruct((T, H), jnp.float32),
        scratch_types=[
            pltpu.VMEM((chunk,), jnp.int32),
            pltpu.VMEM((chunk,), jnp.int32),
            pltpu.VMEM((chunk, H), jnp.float32),
            pltpu.VMEM((chunk, H), jnp.float32),
            pltpu.SemaphoreType.DMA,
            pltpu.SemaphoreType.DMA,
        ],
    )
    def combine(eo_h, d1_h, d2_h, out_h, i1_v, i2_v, a_v, b_v, s1, s2):
        wid = lax.axis_index("s") * 2 + lax.axis_index("c")
        for ch in range(per_w // chunk):
            base = wid * per_w + ch * chunk
            pltpu.sync_copy(d1_h.at[pl.ds(base, chunk)], i1_v)
            pltpu.sync_copy(d2_h.at[pl.ds(base, chunk)], i2_v)
            cp1 = pltpu.async_copy(eo_h.at[i1_v], a_v, s1)
            cp2 = pltpu.async_copy(eo_h.at[i2_v], b_v, s2)
            cp1.wait()
            cp2.wait()
            for r in range(chunk):
                def add_row(cc, _, r=r):
                    sl = pl.ds(cc * 16, 16)
                    a_v[r, sl] = a_v[r, sl] + b_v[r, sl]
                    return 0

                lax.fori_loop(0, H // 16, add_row, 0)
            pltpu.sync_copy(a_v, out_h.at[pl.ds(base, chunk)])

    return combine


def kernel(hidden_states, W_gate, W_gateproj, W_up, W_down):
    B, Sq, H = hidden_states.shape
    T = B * Sq
    E = W_gate.shape[0]
    F = W_up.shape[1]
    capacity = int(math.floor(2 * 1.0 * T / E))
    capacity += capacity % 2
    C = max(capacity, 4)
    S = E * C
    nw = 32

    tokens = hidden_states.reshape(T, H).astype(jnp.float32)
    dest1, dest2, w1, w2 = _router(tokens, W_gate, C)
    src, slotw = _make_invert(T, S)(dest1, dest2, w1, w2)
    disp = _make_dispatch(T, H, S, nw)(tokens, src)
    eo = _ffn(disp, W_gateproj, W_up, W_down, slotw, E, C, H, F)
    ans = _make_combine(T, H, S, nw)(eo, dest1, dest2)
    return ans.reshape(B, Sq, H)


# speedup vs baseline: 2.0457x; 2.0457x over previous
import functools
import math

import jax
import jax.numpy as jnp
from jax import lax
from jax.experimental import pallas as pl
from jax.experimental.pallas import tpu as pltpu
from jax.experimental.pallas import tpu_sc as plsc


def _router_body(C, tok_ref, wg_ref, dest1_ref, dest2_ref, w1_ref, w2_ref):
    T, H = tok_ref.shape
    E = wg_ref.shape[0]
    logits = lax.dot_general(
        tok_ref[...].astype(jnp.bfloat16),
        wg_ref[...].astype(jnp.bfloat16),
        (((1,), (1,)), ((), ())),
        preferred_element_type=jnp.float32,
    )
    m = jnp.max(logits, axis=1, keepdims=True)
    ex = jnp.exp(logits - m)
    probs = ex / jnp.sum(ex, axis=1, keepdims=True)

    eio = lax.broadcasted_iota(jnp.int32, (T, E), 1)
    p1 = jnp.max(probs, axis=1, keepdims=True)
    top1 = jnp.min(jnp.where(probs == p1, eio, E), axis=1, keepdims=True)
    mask1 = eio == top1
    pm = jnp.where(mask1, -jnp.inf, probs)
    p2 = jnp.max(pm, axis=1, keepdims=True)
    top2 = jnp.min(jnp.where(pm == p2, eio, E), axis=1, keepdims=True)
    mask2 = eio == top2

    c = jnp.concatenate(
        [mask1.astype(jnp.int32), mask2.astype(jnp.int32)], axis=1
    )
    k = 1
    while k < T:
        shifted = jnp.concatenate(
            [jnp.zeros((k, 2 * E), jnp.int32), c[: T - k, :]], axis=0
        )
        c = c + shifted
        k *= 2
    csum1, csum2 = c[:, :E], c[:, E:]
    cnt1 = c[T - 1 : T, :E]
    cnt2 = c[T - 1 : T, E:]
    rank1 = csum1 - 1
    rank2 = csum2 - 1 + cnt1
    keep1 = mask1 & (rank1 < C)
    keep2 = mask2 & (rank2 < C)

    r1 = jnp.sum(jnp.where(keep1, rank1, 0), axis=1)
    r2 = jnp.sum(jnp.where(keep2, rank2, 0), axis=1)
    k1 = jnp.sum(keep1.astype(jnp.int32), axis=1)
    k2 = jnp.sum(keep2.astype(jnp.int32), axis=1)
    w1_ref[...] = jnp.sum(jnp.where(keep1, probs, 0.0), axis=1)
    w2_ref[...] = jnp.sum(jnp.where(keep2, probs, 0.0), axis=1)

    tot = cnt1 + cnt2
    eio_r = lax.broadcasted_iota(jnp.int32, (1, E), 1)
    ze = jnp.min(jnp.where(tot < C, eio_r, E), axis=1)
    z = jnp.where(ze < E, ze * C + (C - 1), 0)

    t1 = jnp.sum(jnp.where(mask1, eio, 0), axis=1)
    t2 = jnp.sum(jnp.where(mask2, eio, 0), axis=1)
    dest1_ref[...] = jnp.where(k1 > 0, t1 * C + r1, z)
    dest2_ref[...] = jnp.where(k2 > 0, t2 * C + r2, z)


def _router(tokens, W_gate, C):
    T = tokens.shape[0]
    return pl.pallas_call(
        functools.partial(_router_body, C),
        out_shape=[
            jax.ShapeDtypeStruct((T,), jnp.int32),
            jax.ShapeDtypeStruct((T,), jnp.int32),
            jax.ShapeDtypeStruct((T,), jnp.float32),
            jax.ShapeDtypeStruct((T,), jnp.float32),
        ],
    )(tokens, W_gate)


def _make_invert(T, S):
    mesh = plsc.VectorSubcoreMesh(core_axis_name="c", subcore_axis_name="s")

    @functools.partial(
        pl.kernel,
        mesh=mesh,
        out_type=[
            jax.ShapeDtypeStruct((S,), jnp.int32),
            jax.ShapeDtypeStruct((S,), jnp.float32),
        ],
        scratch_types=[
            pltpu.VMEM((T,), jnp.int32),
            pltpu.VMEM((T,), jnp.int32),
            pltpu.VMEM((T,), jnp.float32),
            pltpu.VMEM((T,), jnp.float32),
            pltpu.VMEM((S,), jnp.int32),
            pltpu.VMEM((S,), jnp.float32),
        ],
        compiler_params=pltpu.CompilerParams(needs_layout_passes=False),
    )
    def invert(d1_h, d2_h, w1_h, w2_h, src_h, slotw_h,
               d1_v, d2_v, w1_v, w2_v, src_v, slotw_v):
        cid = lax.axis_index("c")
        sid = lax.axis_index("s")

        @pl.when(jnp.logical_and(cid == 0, sid == 0))
        def _():
            pltpu.sync_copy(d1_h, d1_v)
            pltpu.sync_copy(d2_h, d2_v)
            pltpu.sync_copy(w1_h, w1_v)
            pltpu.sync_copy(w2_h, w2_v)

            def init(i, _):
                src_v[pl.ds(i * 16, 16)] = jnp.zeros((16,), jnp.int32)
                slotw_v[pl.ds(i * 16, 16)] = jnp.zeros((16,), jnp.float32)
                return 0

            lax.fori_loop(0, S // 16, init, 0)

            def scat(i, _):
                ids = jnp.full((16,), i * 16, jnp.int32) + lax.iota(jnp.int32, 16)
                d1 = d1_v[pl.ds(i * 16, 16)]
                plsc.store_scatter(src_v, [d1], ids)
                plsc.store_scatter(slotw_v, [d1], w1_v[pl.ds(i * 16, 16)])
                d2 = d2_v[pl.ds(i * 16, 16)]
                plsc.store_scatter(src_v, [d2], ids)
                plsc.store_scatter(slotw_v, [d2], w2_v[pl.ds(i * 16, 16)])
                return 0

            lax.fori_loop(0, T // 16, scat, 0)
            pltpu.sync_copy(src_v, src_h)
            pltpu.sync_copy(slotw_v, slotw_h)

    return invert


def _make_dispatch(T, H, S, nw):
    mesh = plsc.VectorSubcoreMesh(core_axis_name="c", subcore_axis_name="s")
    per_w = S // nw
    chunk = 64

    @functools.partial(
        pl.kernel,
        mesh=mesh,
        out_type=jax.ShapeDtypeStruct((S, H), jnp.float32),
        scratch_types=[
            pltpu.VMEM((chunk,), jnp.int32),
            pltpu.VMEM((chunk, H), jnp.float32),
            pltpu.SemaphoreType.DMA,
        ],
    )
    def dispatch(tok_h, src_h, disp_h, idx_v, rows_v, sem):
        wid = lax.axis_index("s") * 2 + lax.axis_index("c")
        base = wid * per_w
        for ch in range(per_w // chunk):
            off = base + ch * chunk
            pltpu.sync_copy(src_h.at[pl.ds(off, chunk)], idx_v)
            pltpu.async_copy(tok_h.at[idx_v], rows_v, sem).wait()
            pltpu.sync_copy(rows_v, disp_h.at[pl.ds(off, chunk)])

    return dispatch


def _ffn_body(NF, x_ref, wg1_ref, wg2_ref, wu_ref, wd_ref, sw_ref, out_ref,
              acc_ref):
    f = pl.program_id(1)

    @pl.when(f == 0)
    def _():
        acc_ref[...] = jnp.zeros_like(acc_ref)

    x = x_ref[0].astype(jnp.bfloat16)
    cd = (((1,), (1,)), ((), ()))
    g1 = lax.dot_general(x, wg1_ref[0].astype(jnp.bfloat16), cd,
                         preferred_element_type=jnp.float32)
    g2 = lax.dot_general(x, wg2_ref[0].astype(jnp.bfloat16), cd,
                         preferred_element_type=jnp.float32)
    u = lax.dot_general(x, wu_ref[0].astype(jnp.bfloat16), cd,
                        preferred_element_type=jnp.float32)
    a = (g1 * (g2 * jax.nn.sigmoid(g2)) * u).astype(jnp.bfloat16)
    acc_ref[...] += lax.dot_general(a, wd_ref[0].astype(jnp.bfloat16), cd,
                                    preferred_element_type=jnp.float32)

    @pl.when(f == NF - 1)
    def _():
        out_ref[0] = acc_ref[...] * sw_ref[0]


def _ffn(disp, W_gateproj, W_up, W_down, slotw, E, C, H, F):
    Fb = 512
    NF = F // Fb
    x3 = disp.reshape(E, C, H)
    sw3 = slotw.reshape(E, C, 1)
    out = pl.pallas_call(
        functools.partial(_ffn_body, NF),
        grid=(E, NF),
        in_specs=[
            pl.BlockSpec((1, C, H), lambda e, f: (e, 0, 0)),
            pl.BlockSpec((1, Fb, H), lambda e, f: (e, f, 0)),
            pl.BlockSpec((1, Fb, H), lambda e, f, NF=NF: (e, f + NF, 0)),
            pl.BlockSpec((1, Fb, H), lambda e, f: (e, f, 0)),
            pl.BlockSpec((1, H, Fb), lambda e, f: (e, 0, f)),
            pl.BlockSpec((1, C, 1), lambda e, f: (e, 0, 0)),
        ],
        out_specs=pl.BlockSpec((1, C, H), lambda e, f: (e, 0, 0)),
        out_shape=jax.ShapeDtypeStruct((E, C, H), jnp.float32),
        scratch_shapes=[pltpu.VMEM((C, H), jnp.float32)],
    )(x3, W_gateproj, W_gateproj, W_up, W_down, sw3)
    return out.reshape(E * C, H)


def _make_combine(T, H, S, nw):
    mesh = plsc.VectorSubcoreMesh(core_axis_name="c", subcore_axis_name="s")
    per_w = T // nw
    chunk = 32

    @functools.partial(
        pl.kernel,
        mesh=mesh,
        out_type=jax.ShapeDtypeStruct((T, H), jnp.float32),
        scratch_types=[
            pltpu.VMEM((chunk,), jnp.int32),
            pltpu.VMEM((chunk,), jnp.int32),
            pltpu.VMEM((chunk, H), jnp.float32),
            pltpu.VMEM((chunk, H), jnp.float32),
            pltpu.SemaphoreType.DMA,
            pltpu.SemaphoreType.DMA,
        ],
    )
    def combine(eo_h, d1_h, d2_h, out_h, i1_v, i2_v, a_v, b_v, s1, s2):
        wid = lax.axis_index("s") * 2 + lax.axis_index("c")
        for ch in range(per_w // chunk):
            base = wid * per_w + ch * chunk
            pltpu.sync_copy(d1_h.at[pl.ds(base, chunk)], i1_v)
            pltpu.sync_copy(d2_h.at[pl.ds(base, chunk)], i2_v)
            cp1 = pltpu.async_copy(eo_h.at[i1_v], a_v, s1)
            cp2 = pltpu.async_copy(eo_h.at[i2_v], b_v, s2)
            cp1.wait()
            cp2.wait()
            for r in range(chunk):
                def add_row(cc, _, r=r):
                    sl = pl.ds(cc * 16, 16)
                    a_v[r, sl] = a_v[r, sl] + b_v[r, sl]
                    return 0

                lax.fori_loop(0, H // 16, add_row, 0)
            pltpu.sync_copy(a_v, out_h.at[pl.ds(base, chunk)])

    return combine


def kernel(hidden_states, W_gate, W_gateproj, W_up, W_down):
    B, Sq, H = hidden_states.shape
    T = B * Sq
    E = W_gate.shape[0]
    F = W_up.shape[1]
    capacity = int(math.floor(2 * 1.0 * T / E))
    capacity += capacity % 2
    C = max(capacity, 4)
    S = E * C
    nw = 32

    tokens = hidden_states.reshape(T, H).astype(jnp.float32)
    dest1, dest2, w1, w2 = _router(tokens, W_gate, C)
    src, slotw = _make_invert(T, S)(dest1, dest2, w1, w2)
    disp = _make_dispatch(T, H, S, nw)(tokens, src)
    eo = _ffn(disp, W_gateproj, W_up, W_down, slotw, E, C, H, F)
    ans = _make_combine(T, H, S, nw)(eo, dest1, dest2)
    return ans.reshape(B, Sq, H)


# speedup vs baseline: 2.0990x; 1.0260x over previous
import functools
import math

import jax
import jax.numpy as jnp
from jax import lax
from jax.experimental import pallas as pl
from jax.experimental.pallas import tpu as pltpu
from jax.experimental.pallas import tpu_sc as plsc


def _router_body(C, tok_ref, wg_ref, dest1_ref, dest2_ref, w1_ref, w2_ref):
    T, H = tok_ref.shape
    E = wg_ref.shape[0]
    logits = lax.dot_general(
        tok_ref[...].astype(jnp.bfloat16),
        wg_ref[...].astype(jnp.bfloat16),
        (((1,), (1,)), ((), ())),
        preferred_element_type=jnp.float32,
    )
    m = jnp.max(logits, axis=1, keepdims=True)
    ex = jnp.exp(logits - m)
    probs = ex / jnp.sum(ex, axis=1, keepdims=True)

    eio = lax.broadcasted_iota(jnp.int32, (T, E), 1)
    p1 = jnp.max(probs, axis=1, keepdims=True)
    top1 = jnp.min(jnp.where(probs == p1, eio, E), axis=1, keepdims=True)
    mask1 = eio == top1
    pm = jnp.where(mask1, -jnp.inf, probs)
    p2 = jnp.max(pm, axis=1, keepdims=True)
    top2 = jnp.min(jnp.where(pm == p2, eio, E), axis=1, keepdims=True)
    mask2 = eio == top2

    c = jnp.concatenate(
        [mask1.astype(jnp.int32), mask2.astype(jnp.int32)], axis=1
    )
    k = 1
    while k < T:
        shifted = jnp.concatenate(
            [jnp.zeros((k, 2 * E), jnp.int32), c[: T - k, :]], axis=0
        )
        c = c + shifted
        k *= 2
    csum1, csum2 = c[:, :E], c[:, E:]
    cnt1 = c[T - 1 : T, :E]
    cnt2 = c[T - 1 : T, E:]
    rank1 = csum1 - 1
    rank2 = csum2 - 1 + cnt1
    keep1 = mask1 & (rank1 < C)
    keep2 = mask2 & (rank2 < C)

    r1 = jnp.sum(jnp.where(keep1, rank1, 0), axis=1)
    r2 = jnp.sum(jnp.where(keep2, rank2, 0), axis=1)
    k1 = jnp.sum(keep1.astype(jnp.int32), axis=1)
    k2 = jnp.sum(keep2.astype(jnp.int32), axis=1)
    w1_ref[...] = jnp.sum(jnp.where(keep1, probs, 0.0), axis=1)
    w2_ref[...] = jnp.sum(jnp.where(keep2, probs, 0.0), axis=1)

    tot = cnt1 + cnt2
    eio_r = lax.broadcasted_iota(jnp.int32, (1, E), 1)
    ze = jnp.min(jnp.where(tot < C, eio_r, E), axis=1)
    z = jnp.where(ze < E, ze * C + (C - 1), 0)

    t1 = jnp.sum(jnp.where(mask1, eio, 0), axis=1)
    t2 = jnp.sum(jnp.where(mask2, eio, 0), axis=1)
    dest1_ref[...] = jnp.where(k1 > 0, t1 * C + r1, z)
    dest2_ref[...] = jnp.where(k2 > 0, t2 * C + r2, z)


def _router(tokens, W_gate, C):
    T = tokens.shape[0]
    return pl.pallas_call(
        functools.partial(_router_body, C),
        out_shape=[
            jax.ShapeDtypeStruct((T,), jnp.int32),
            jax.ShapeDtypeStruct((T,), jnp.int32),
            jax.ShapeDtypeStruct((T,), jnp.float32),
            jax.ShapeDtypeStruct((T,), jnp.float32),
        ],
    )(tokens, W_gate)


def _make_dispatch(T, H, S, nw):
    mesh = plsc.VectorSubcoreMesh(core_axis_name="c", subcore_axis_name="s")
    per_w = S // nw
    chunk = 32
    nch = per_w // chunk

    @functools.partial(
        pl.kernel,
        mesh=mesh,
        out_type=[
            jax.ShapeDtypeStruct((S, H), jnp.float32),
            jax.ShapeDtypeStruct((S,), jnp.float32),
        ],
        scratch_types=[
            pltpu.VMEM((T,), jnp.int32),
            pltpu.VMEM((T,), jnp.int32),
            pltpu.VMEM((T,), jnp.float32),
            pltpu.VMEM((T,), jnp.float32),
            pltpu.VMEM((per_w,), jnp.int32),
            pltpu.VMEM((per_w,), jnp.float32),
            pltpu.VMEM((chunk,), jnp.int32),
            pltpu.VMEM((chunk,), jnp.int32),
            pltpu.VMEM((chunk, H), jnp.float32),
            pltpu.VMEM((chunk, H), jnp.float32),
            pltpu.SemaphoreType.DMA,
            pltpu.SemaphoreType.DMA,
        ],
        compiler_params=pltpu.CompilerParams(needs_layout_passes=False),
    )
    def dispatch(tok_h, d1_h, d2_h, w1_h, w2_h, disp_h, slotw_h,
                 d1_v, d2_v, w1_v, w2_v, src_v, slotw_v,
                 ia_v, ib_v, ra_v, rb_v, sa, sb):
        wid = lax.axis_index("s") * 2 + lax.axis_index("c")
        base = wid * per_w
        pltpu.sync_copy(d1_h, d1_v)
        pltpu.sync_copy(d2_h, d2_v)
        pltpu.sync_copy(w1_h, w1_v)
        pltpu.sync_copy(w2_h, w2_v)

        def init(i, _):
            src_v[pl.ds(i * 16, 16)] = jnp.zeros((16,), jnp.int32)
            slotw_v[pl.ds(i * 16, 16)] = jnp.zeros((16,), jnp.float32)
            return 0

        lax.fori_loop(0, per_w // 16, init, 0)

        def scat(i, _):
            ids = jnp.full((16,), i * 16, jnp.int32) + lax.iota(jnp.int32, 16)
            d1 = d1_v[pl.ds(i * 16, 16)] - base
            m1 = (d1 >= 0) & (d1 < per_w)
            plsc.store_scatter(src_v, [d1], ids, mask=m1)
            plsc.store_scatter(slotw_v, [d1], w1_v[pl.ds(i * 16, 16)], mask=m1)
            d2 = d2_v[pl.ds(i * 16, 16)] - base
            m2 = (d2 >= 0) & (d2 < per_w)
            plsc.store_scatter(src_v, [d2], ids, mask=m2)
            plsc.store_scatter(slotw_v, [d2], w2_v[pl.ds(i * 16, 16)], mask=m2)
            return 0

        lax.fori_loop(0, T // 16, scat, 0)
        pltpu.sync_copy(slotw_v, slotw_h.at[pl.ds(base, per_w)])

        def idx_chunk(ch, dst):
            def cp(i, _):
                dst[pl.ds(i * 16, 16)] = src_v[pl.ds(ch * chunk + i * 16, 16)]
                return 0
            lax.fori_loop(0, chunk // 16, cp, 0)

        idx_chunk(0, ia_v)
        cpa = pltpu.async_copy(tok_h.at[ia_v], ra_v, sa)
        for ch in range(nch):
            nxt = ch + 1
            if nxt < nch:
                if nxt % 2 == 1:
                    idx_chunk(nxt, ib_v)
                    cpb = pltpu.async_copy(tok_h.at[ib_v], rb_v, sb)
                else:
                    idx_chunk(nxt, ia_v)
                    cpa = pltpu.async_copy(tok_h.at[ia_v], ra_v, sa)
            if ch % 2 == 0:
                cpa.wait()
                pltpu.sync_copy(ra_v, disp_h.at[pl.ds(base + ch * chunk, chunk)])
            else:
                cpb.wait()
                pltpu.sync_copy(rb_v, disp_h.at[pl.ds(base + ch * chunk, chunk)])

    return dispatch


def _ffn_body(NF, x_ref, wg1_ref, wg2_ref, wu_ref, wd_ref, sw_ref, out_ref,
              acc_ref):
    f = pl.program_id(1)

    @pl.when(f == 0)
    def _():
        acc_ref[...] = jnp.zeros_like(acc_ref)

    x = x_ref[0].astype(jnp.bfloat16)
    cd = (((1,), (1,)), ((), ()))
    g1 = lax.dot_general(x, wg1_ref[0].astype(jnp.bfloat16), cd,
                         preferred_element_type=jnp.float32)
    g2 = lax.dot_general(x, wg2_ref[0].astype(jnp.bfloat16), cd,
                         preferred_element_type=jnp.float32)
    u = lax.dot_general(x, wu_ref[0].astype(jnp.bfloat16), cd,
                        preferred_element_type=jnp.float32)
    a = (g1 * (g2 * jax.nn.sigmoid(g2)) * u).astype(jnp.bfloat16)
    acc_ref[...] += lax.dot_general(a, wd_ref[0].astype(jnp.bfloat16), cd,
                                    preferred_element_type=jnp.float32)

    @pl.when(f == NF - 1)
    def _():
        out_ref[0] = acc_ref[...] * sw_ref[0]


def _ffn(disp, W_gateproj, W_up, W_down, slotw, E, C, H, F):
    Fb = 512
    NF = F // Fb
    x3 = disp.reshape(E, C, H)
    sw3 = slotw.reshape(E, C, 1)
    out = pl.pallas_call(
        functools.partial(_ffn_body, NF),
        grid=(E, NF),
        in_specs=[
            pl.BlockSpec((1, C, H), lambda e, f: (e, 0, 0)),
            pl.BlockSpec((1, Fb, H), lambda e, f: (e, f, 0)),
            pl.BlockSpec((1, Fb, H), lambda e, f, NF=NF: (e, f + NF, 0)),
            pl.BlockSpec((1, Fb, H), lambda e, f: (e, f, 0)),
            pl.BlockSpec((1, H, Fb), lambda e, f: (e, 0, f)),
            pl.BlockSpec((1, C, 1), lambda e, f: (e, 0, 0)),
        ],
        out_specs=pl.BlockSpec((1, C, H), lambda e, f: (e, 0, 0)),
        out_shape=jax.ShapeDtypeStruct((E, C, H), jnp.float32),
        scratch_shapes=[pltpu.VMEM((C, H), jnp.float32)],
    )(x3, W_gateproj, W_gateproj, W_up, W_down, sw3)
    return out.reshape(E * C, H)


def _make_combine(T, H, S, nw):
    mesh = plsc.VectorSubcoreMesh(core_axis_name="c", subcore_axis_name="s")
    per_w = T // nw
    chunk = 16

    nch = per_w // chunk

    @functools.partial(
        pl.kernel,
        mesh=mesh,
        out_type=jax.ShapeDtypeStruct((T, H), jnp.float32),
        scratch_types=[
            pltpu.VMEM((per_w,), jnp.int32),
            pltpu.VMEM((per_w,), jnp.int32),
            pltpu.VMEM((chunk, H), jnp.float32),
            pltpu.VMEM((chunk, H), jnp.float32),
            pltpu.VMEM((chunk, H), jnp.float32),
            pltpu.VMEM((chunk, H), jnp.float32),
            pltpu.SemaphoreType.DMA,
            pltpu.SemaphoreType.DMA,
        ],
        compiler_params=pltpu.CompilerParams(needs_layout_passes=False),
    )
    def combine(eo_h, d1_h, d2_h, out_h, i1_v, i2_v, a0_v, b0_v, a1_v, b1_v,
                s0, s1):
        wid = lax.axis_index("s") * 2 + lax.axis_index("c")
        base = wid * per_w
        pltpu.sync_copy(d1_h.at[pl.ds(base, per_w)], i1_v)
        pltpu.sync_copy(d2_h.at[pl.ds(base, per_w)], i2_v)
        bufs = [(a0_v, b0_v, s0), (a1_v, b1_v, s1)]
        pend = [None, None]

        def issue(ch):
            a_v, b_v, sem = bufs[ch % 2]
            sl = pl.ds(ch * chunk, chunk)
            c1 = pltpu.async_copy(eo_h.at[i1_v.at[sl]], a_v, sem)
            c2 = pltpu.async_copy(eo_h.at[i2_v.at[sl]], b_v, sem)
            pend[ch % 2] = (c1, c2)

        issue(0)
        for ch in range(nch):
            if ch + 1 < nch:
                issue(ch + 1)
            a_v, b_v, _ = bufs[ch % 2]
            c1, c2 = pend[ch % 2]
            c1.wait()
            c2.wait()
            for r in range(chunk):
                def add_row(cc, _, r=r):
                    for j in range(4):
                        sl = pl.ds(cc * 64 + j * 16, 16)
                        plsc.addupdate(a_v.at[r, sl], b_v[r, sl])
                    return 0

                lax.fori_loop(0, H // 64, add_row, 0)
            pltpu.sync_copy(a_v, out_h.at[pl.ds(base + ch * chunk, chunk)])

    return combine


def kernel(hidden_states, W_gate, W_gateproj, W_up, W_down):
    B, Sq, H = hidden_states.shape
    T = B * Sq
    E = W_gate.shape[0]
    F = W_up.shape[1]
    capacity = int(math.floor(2 * 1.0 * T / E))
    capacity += capacity % 2
    C = max(capacity, 4)
    S = E * C
    nw = 32

    tokens = hidden_states.reshape(T, H).astype(jnp.float32)
    dest1, dest2, w1, w2 = _router(tokens, W_gate, C)
    disp, slotw = _make_dispatch(T, H, S, nw)(tokens, dest1, dest2, w1, w2)
    eo = _ffn(disp, W_gateproj, W_up, W_down, slotw, E, C, H, F)
    ans = _make_combine(T, H, S, nw)(eo, dest1, dest2)
    return ans.reshape(B, Sq, H)
